# baseline (speedup 1.0000x reference)
import functools

import jax
import jax.numpy as jnp
from jax import lax
from jax.experimental import pallas as pl
from jax.experimental.pallas import tpu as pltpu
from jax.experimental.pallas import tpu_sc as plsc

_N_EDGES = 320000
_D_EDGE = 16
_TC_ROWS = 160000
_SC_ROWS = _N_EDGES - _TC_ROWS
_NC = 2
_NS = 16
_NW = _NC * _NS
_ROWS_PER_W = _SC_ROWS // _NW
_CHUNK = 200
_NCHUNK = _ROWS_PER_W // _CHUNK

_TC_GRID = 80
_NODE_STEPS = 10


@functools.partial(
    pl.kernel,
    mesh=plsc.VectorSubcoreMesh(core_axis_name="c", subcore_axis_name="s"),
    out_type=jax.ShapeDtypeStruct((_SC_ROWS, _D_EDGE), jnp.float32),
    scratch_types=[
        pltpu.VMEM((_CHUNK, _D_EDGE), jnp.float32),
        pltpu.VMEM((_CHUNK, _D_EDGE), jnp.float32),
        pltpu.SemaphoreType.DMA,
        pltpu.SemaphoreType.DMA,
        pltpu.SemaphoreType.DMA,
        pltpu.SemaphoreType.DMA,
    ],
)
def _edge_copy_sc(edge_hbm, out_hbm, buf0, buf1, gs0, gs1, ss0, ss1):
    wid = lax.axis_index("s") * _NC + lax.axis_index("c")
    base = _TC_ROWS + wid * _ROWS_PER_W

    bufs = (buf0, buf1)
    gsems = (gs0, gs1)
    ssems = (ss0, ss1)

    def src(k):
        return edge_hbm.at[pl.ds(base + k * _CHUNK, _CHUNK), :]

    def dst(k):
        return out_hbm.at[pl.ds(base - _TC_ROWS + k * _CHUNK, _CHUNK), :]

    gathers = [None] * _NCHUNK
    scatters = [None] * _NCHUNK
    gathers[0] = pltpu.async_copy(src(0), bufs[0], gsems[0])
    for k in range(_NCHUNK):
        b = k % 2
        gathers[k].wait()
        scatters[k] = pltpu.async_copy(bufs[b], dst(k), ssems[b])
        if k + 1 < _NCHUNK:
            if k >= 1:
                scatters[k - 1].wait()
            nb = (k + 1) % 2
            gathers[k + 1] = pltpu.async_copy(src(k + 1), bufs[nb], gsems[nb])
    scatters[_NCHUNK - 2].wait()
    scatters[_NCHUNK - 1].wait()


def _tc_copy_body(node_ref, edge_ref, node_out_ref, edge_out_ref):
    edge_out_ref[...] = edge_ref[...]

    @pl.when(pl.program_id(0) < _NODE_STEPS)
    def _():
        node_out_ref[...] = node_ref[...]


def kernel(node_feats, edge_index, edge_attr):
    n_nodes, d_feat = node_feats.shape
    nb = n_nodes // _NODE_STEPS
    eb = _TC_ROWS // _TC_GRID

    def node_idx(i):
        return (jnp.minimum(i, _NODE_STEPS - 1), 0)

    edge_hi_out = _edge_copy_sc(edge_attr)

    node_out, edge_lo_out = pl.pallas_call(
        _tc_copy_body,
        grid=(_TC_GRID,),
        in_specs=[
            pl.BlockSpec((nb, d_feat), node_idx),
            pl.BlockSpec((eb, _D_EDGE), lambda i: (i, 0)),
        ],
        out_specs=[
            pl.BlockSpec((nb, d_feat), node_idx),
            pl.BlockSpec((eb, _D_EDGE), lambda i: (i, 0)),
        ],
        out_shape=[
            jax.ShapeDtypeStruct((n_nodes, d_feat), node_feats.dtype),
            jax.ShapeDtypeStruct((_TC_ROWS, _D_EDGE), edge_attr.dtype),
        ],
    )(node_feats, edge_attr)
    edge_out = jnp.concatenate([edge_lo_out, edge_hi_out], axis=0)
    return (node_out, edge_out)


# speedup vs baseline: 1.1772x; 1.1772x over previous
import jax
import jax.numpy as jnp
from jax.experimental import pallas as pl

_GRID = 40
_NODE_STEPS = 10


def _copy_body(node_ref, edge_ref, node_out_ref, edge_out_ref):
    edge_out_ref[...] = edge_ref[...]

    @pl.when(pl.program_id(0) < _NODE_STEPS)
    def _():
        node_out_ref[...] = node_ref[...]


def kernel(node_feats, edge_index, edge_attr):
    n_nodes, d_feat = node_feats.shape
    n_edges, d_edge = edge_attr.shape
    nb = n_nodes // _NODE_STEPS
    eb = n_edges // _GRID

    def node_idx(i):
        return (jnp.minimum(i, _NODE_STEPS - 1), 0)

    node_out, edge_out = pl.pallas_call(
        _copy_body,
        grid=(_GRID,),
        in_specs=[
            pl.BlockSpec((nb, d_feat), node_idx),
            pl.BlockSpec((eb, d_edge), lambda i: (i, 0)),
        ],
        out_specs=[
            pl.BlockSpec((nb, d_feat), node_idx),
            pl.BlockSpec((eb, d_edge), lambda i: (i, 0)),
        ],
        out_shape=[
            jax.ShapeDtypeStruct((n_nodes, d_feat), node_feats.dtype),
            jax.ShapeDtypeStruct((n_edges, d_edge), edge_attr.dtype),
        ],
    )(node_feats, edge_attr)
    return (node_out, edge_out)


# speedup vs baseline: 1.1835x; 1.0054x over previous
import jax
import jax.numpy as jnp
from jax.experimental import pallas as pl

_GRID = 20
_NODE_STEPS = 10


def _copy_body(node_ref, edge_ref, node_out_ref, edge_out_ref):
    edge_out_ref[...] = edge_ref[...]

    @pl.when(pl.program_id(0) < _NODE_STEPS)
    def _():
        node_out_ref[...] = node_ref[...]


def kernel(node_feats, edge_index, edge_attr):
    n_nodes, d_feat = node_feats.shape
    n_edges, d_edge = edge_attr.shape
    nb = n_nodes // _NODE_STEPS
    eb = n_edges // _GRID

    def node_idx(i):
        return (jnp.minimum(i, _NODE_STEPS - 1), 0)

    node_out, edge_out = pl.pallas_call(
        _copy_body,
        grid=(_GRID,),
        in_specs=[
            pl.BlockSpec((nb, d_feat), node_idx),
            pl.BlockSpec((eb, d_edge), lambda i: (i, 0)),
        ],
        out_specs=[
            pl.BlockSpec((nb, d_feat), node_idx),
            pl.BlockSpec((eb, d_edge), lambda i: (i, 0)),
        ],
        out_shape=[
            jax.ShapeDtypeStruct((n_nodes, d_feat), node_feats.dtype),
            jax.ShapeDtypeStruct((n_edges, d_edge), edge_attr.dtype),
        ],
    )(node_feats, edge_attr)
    return (node_out, edge_out)
